# baseline (speedup 1.0000x reference)
import functools

import jax
import jax.numpy as jnp
from jax import lax
from jax.experimental import pallas as pl
from jax.experimental.pallas import tpu as pltpu
from jax.experimental.pallas import tpu_sc as plsc

HEAD_DIM = 128
CHUNK = 128
NBUF = 3
TC_ROWS = 1024
COLS = 8


def _sc_gather_fn(B, S, NC, NS):
    mesh = plsc.VectorSubcoreMesh(core_axis_name="c", subcore_axis_name="s")
    N = B * S
    NW = NC * NS
    per_w = N // NW
    blocks = S // per_w
    chunks_per_w = per_w // CHUNK

    @functools.partial(
        pl.kernel,
        mesh=mesh,
        out_type=jax.ShapeDtypeStruct((N, HEAD_DIM), jnp.float32),
        scratch_types=[
            pltpu.VMEM((per_w,), jnp.int32),
            pltpu.VMEM((NBUF, CHUNK, HEAD_DIM), jnp.float32),
        ]
        + [pltpu.SemaphoreType.DMA] * (2 * NBUF),
    )
    def body(idx_hbm, tab_hbm, out_hbm, idx_v, row_v, *sems):
        gsem, wsem = sems[:NBUF], sems[NBUF:]
        wid = lax.axis_index("s") * NC + lax.axis_index("c")
        batch = wid // blocks
        col0 = (wid % blocks) * per_w
        row0 = wid * per_w
        pltpu.sync_copy(idx_hbm.at[batch, pl.ds(col0, per_w)], idx_v)

        def issue_gather(j):
            b = j % NBUF
            ids = idx_v.at[pl.ds(j * CHUNK, CHUNK)]
            return pltpu.async_copy(tab_hbm.at[ids], row_v.at[b], gsem[b])

        pending_g = [None] * NBUF
        pending_w = [None] * NBUF
        for j in range(min(NBUF - 1, chunks_per_w)):
            pending_g[j % NBUF] = issue_gather(j)
        for j in range(chunks_per_w):
            b = j % NBUF
            jn = j + NBUF - 1
            if jn < chunks_per_w:
                nb = jn % NBUF
                if pending_w[nb] is not None:
                    pending_w[nb].wait()
                    pending_w[nb] = None
                pending_g[nb] = issue_gather(jn)
            pending_g[b].wait()
            pending_g[b] = None
            base = row0 + j * CHUNK
            pending_w[b] = pltpu.async_copy(
                row_v.at[b], out_hbm.at[pl.ds(base, CHUNK)], wsem[b])
        for w in pending_w:
            if w is not None:
                w.wait()

    return body


def _tc_cos_body(pos_ref, cqs_ref, crs_ref, out_ref):
    i = pl.program_id(0)
    cb = i // (128 // COLS)
    pos_blk = pos_ref[:, pl.ds(pl.multiple_of(cb * 128, 128), 128)]
    col_ids = lax.broadcasted_iota(jnp.int32, (128, COLS), 1).astype(
        jnp.float32)
    qiota = lax.broadcasted_iota(jnp.int32, (128, 256), 1).astype(jnp.float32)
    riota = lax.broadcasted_iota(jnp.int32, (128, 128), 1).astype(jnp.float32)
    qbase = (i % (128 // COLS)) * COLS
    sel = (lax.broadcasted_iota(jnp.int32, (128, COLS), 0).astype(jnp.float32)
           == col_ids + qbase).astype(jnp.float32)
    cols = jax.lax.dot_general(
        pos_blk, sel, (((1,), (0,)), ((), ())),
        precision=jax.lax.Precision.HIGHEST)
    for q in range(COLS):
        pos = cols[:, q:q + 1]
        pq = jnp.floor(pos * (1.0 / 128.0))
        pr = pos - 128.0 * pq
        oh_q = (qiota == pq).astype(jnp.bfloat16)
        oh_r = (riota == pr).astype(jnp.bfloat16)
        dot = lambda a, b: jax.lax.dot_general(
            a, b, (((1,), (0,)), ((), ())),
            preferred_element_type=jnp.float32)
        qsel = dot(oh_q, cqs_ref[...])
        rsel = dot(oh_r, crs_ref[...])
        out_ref[pl.ds(q * 128, 128), :] = (
            qsel[:, :HEAD_DIM] * rsel[:, :HEAD_DIM]
            - qsel[:, HEAD_DIM:] * rsel[:, HEAD_DIM:])


def _tc_cos_fn(N):
    ncols = N // 128
    return pl.pallas_call(
        _tc_cos_body,
        grid=(N // TC_ROWS,),
        in_specs=[
            pl.BlockSpec((128, ncols), lambda i: (0, 0)),
            pl.BlockSpec((256, 2 * HEAD_DIM), lambda i: (0, 0)),
            pl.BlockSpec((128, 2 * HEAD_DIM), lambda i: (0, 0)),
        ],
        out_specs=pl.BlockSpec((TC_ROWS, HEAD_DIM), lambda i: (i, 0)),
        out_shape=jax.ShapeDtypeStruct((N, HEAD_DIM), jnp.float32),
    )


def kernel(position_ids, cos_emb, sin_emb):
    B, S = position_ids.shape
    N = B * S
    info = plsc.get_sparse_core_info()
    NC, NS = info.num_cores, info.num_subcores

    idx = position_ids.astype(jnp.int32)
    sin_flat = _sc_gather_fn(B, S, NC, NS)(idx, sin_emb)

    pos_t = idx.reshape(N // 128, 128).T.astype(jnp.float32)
    cqs = jnp.concatenate((cos_emb[::128], sin_emb[::128]),
                          axis=1).astype(jnp.bfloat16)
    crs = jnp.concatenate((cos_emb[:128], sin_emb[:128]),
                          axis=1).astype(jnp.bfloat16)
    cos_flat = _tc_cos_fn(N)(pos_t, cqs, crs)

    return (cos_flat.reshape(B, S, HEAD_DIM),
            sin_flat.reshape(B, S, HEAD_DIM))


# speedup vs baseline: 1.0302x; 1.0302x over previous
import functools

import jax
import jax.numpy as jnp
from jax import lax
from jax.experimental import pallas as pl
from jax.experimental.pallas import tpu as pltpu
from jax.experimental.pallas import tpu_sc as plsc

HEAD_DIM = 128
CHUNK = 128
NBUF = 3
TC_ROWS = 2048
COLS = 16


def _sc_gather_fn(B, S, NC, NS):
    mesh = plsc.VectorSubcoreMesh(core_axis_name="c", subcore_axis_name="s")
    N = B * S
    NW = NC * NS
    per_w = N // NW
    blocks = S // per_w
    chunks_per_w = per_w // CHUNK

    @functools.partial(
        pl.kernel,
        mesh=mesh,
        out_type=jax.ShapeDtypeStruct((N, HEAD_DIM), jnp.float32),
        scratch_types=[
            pltpu.VMEM((per_w,), jnp.int32),
            pltpu.VMEM((NBUF, CHUNK, HEAD_DIM), jnp.float32),
        ]
        + [pltpu.SemaphoreType.DMA] * (2 * NBUF),
    )
    def body(idx_hbm, tab_hbm, out_hbm, idx_v, row_v, *sems):
        gsem, wsem = sems[:NBUF], sems[NBUF:]
        wid = lax.axis_index("s") * NC + lax.axis_index("c")
        batch = wid // blocks
        col0 = (wid % blocks) * per_w
        row0 = wid * per_w
        pltpu.sync_copy(idx_hbm.at[batch, pl.ds(col0, per_w)], idx_v)

        def issue_gather(j):
            b = j % NBUF
            ids = idx_v.at[pl.ds(j * CHUNK, CHUNK)]
            return pltpu.async_copy(tab_hbm.at[ids], row_v.at[b], gsem[b])

        pending_g = [None] * NBUF
        pending_w = [None] * NBUF
        for j in range(min(NBUF - 1, chunks_per_w)):
            pending_g[j % NBUF] = issue_gather(j)
        for j in range(chunks_per_w):
            b = j % NBUF
            jn = j + NBUF - 1
            if jn < chunks_per_w:
                nb = jn % NBUF
                if pending_w[nb] is not None:
                    pending_w[nb].wait()
                    pending_w[nb] = None
                pending_g[nb] = issue_gather(jn)
            pending_g[b].wait()
            pending_g[b] = None
            base = row0 + j * CHUNK
            pending_w[b] = pltpu.async_copy(
                row_v.at[b], out_hbm.at[pl.ds(base, CHUNK)], wsem[b])
        for w in pending_w:
            if w is not None:
                w.wait()

    return body


def _tc_cos_body(pos_ref, cq_ref, sq_ref, cr_ref, sr_ref, out_ref):
    i = pl.program_id(0)
    cb = i // (128 // COLS)
    pos_blk = pos_ref[:, pl.ds(pl.multiple_of(cb * 128, 128), 128)]
    col_ids = lax.broadcasted_iota(jnp.int32, (128, COLS), 1).astype(
        jnp.float32)
    qiota = lax.broadcasted_iota(jnp.int32, (128, 256), 1).astype(jnp.float32)
    riota = lax.broadcasted_iota(jnp.int32, (128, 128), 1).astype(jnp.float32)
    qbase = (i % (128 // COLS)) * COLS
    sel = (lax.broadcasted_iota(jnp.int32, (128, COLS), 0).astype(jnp.float32)
           == col_ids + qbase).astype(jnp.float32)
    cols = jax.lax.dot_general(
        pos_blk, sel, (((1,), (0,)), ((), ())),
        precision=jax.lax.Precision.HIGHEST)
    for q in range(COLS):
        pos = cols[:, q:q + 1]
        pq = jnp.floor(pos * (1.0 / 128.0))
        pr = pos - 128.0 * pq
        oh_q = (qiota == pq).astype(jnp.bfloat16)
        oh_r = (riota == pr).astype(jnp.bfloat16)
        dot = lambda a, b: jax.lax.dot_general(
            a, b, (((1,), (0,)), ((), ())),
            preferred_element_type=jnp.float32)
        ca, sa = dot(oh_q, cq_ref[...]), dot(oh_q, sq_ref[...])
        cb_, sb = dot(oh_r, cr_ref[...]), dot(oh_r, sr_ref[...])
        out_ref[pl.ds(q * 128, 128), :] = ca * cb_ - sa * sb


def _tc_cos_fn(N):
    ncols = N // 128
    return pl.pallas_call(
        _tc_cos_body,
        grid=(N // TC_ROWS,),
        in_specs=[
            pl.BlockSpec((128, ncols), lambda i: (0, 0)),
            pl.BlockSpec((256, HEAD_DIM), lambda i: (0, 0)),
            pl.BlockSpec((256, HEAD_DIM), lambda i: (0, 0)),
            pl.BlockSpec((128, HEAD_DIM), lambda i: (0, 0)),
            pl.BlockSpec((128, HEAD_DIM), lambda i: (0, 0)),
        ],
        out_specs=pl.BlockSpec((TC_ROWS, HEAD_DIM), lambda i: (i, 0)),
        out_shape=jax.ShapeDtypeStruct((N, HEAD_DIM), jnp.float32),
    )


def kernel(position_ids, cos_emb, sin_emb):
    B, S = position_ids.shape
    N = B * S
    info = plsc.get_sparse_core_info()
    NC, NS = info.num_cores, info.num_subcores

    idx = position_ids.astype(jnp.int32)
    sin_flat = _sc_gather_fn(B, S, NC, NS)(idx, sin_emb)

    pos_t = idx.reshape(N // 128, 128).T.astype(jnp.float32)
    cq = cos_emb[::128].astype(jnp.bfloat16)
    sq = sin_emb[::128].astype(jnp.bfloat16)
    cr = cos_emb[:128].astype(jnp.bfloat16)
    sr = sin_emb[:128].astype(jnp.bfloat16)
    cos_flat = _tc_cos_fn(N)(pos_t, cq, sq, cr, sr)

    return (cos_flat.reshape(B, S, HEAD_DIM),
            sin_flat.reshape(B, S, HEAD_DIM))


# speedup vs baseline: 1.8283x; 1.7747x over previous
import functools

import jax
import jax.numpy as jnp
from jax import lax
from jax.experimental import pallas as pl
from jax.experimental.pallas import tpu as pltpu
from jax.experimental.pallas import tpu_sc as plsc

HEAD_DIM = 128
CHUNK = 128
NBUF = 3


def _rope_gather_fn(B, S, NC, NS):
    mesh = plsc.VectorSubcoreMesh(core_axis_name="c", subcore_axis_name="s")
    N = B * S
    NW = NC * NS
    per_w = N // NW
    blocks = S // per_w
    chunks_per_w = per_w // CHUNK

    @functools.partial(
        pl.kernel,
        mesh=mesh,
        out_type=(
            jax.ShapeDtypeStruct((N, HEAD_DIM), jnp.float32),
            jax.ShapeDtypeStruct((N, HEAD_DIM), jnp.float32),
        ),
        scratch_types=[
            pltpu.VMEM((per_w,), jnp.int32),
            pltpu.VMEM((NBUF, CHUNK, HEAD_DIM), jnp.float32),
            pltpu.VMEM((NBUF, CHUNK, HEAD_DIM), jnp.float32),
        ]
        + [pltpu.SemaphoreType.DMA] * (2 * NBUF),
    )
    def body(idx_hbm, cos_hbm, sin_hbm, cos_out, sin_out,
             idx_v, cos_v, sin_v, *sems):
        gsem, wsem = sems[:NBUF], sems[NBUF:]
        wid = lax.axis_index("s") * NC + lax.axis_index("c")
        batch = wid // blocks
        col0 = (wid % blocks) * per_w
        row0 = wid * per_w
        pltpu.sync_copy(idx_hbm.at[batch, pl.ds(col0, per_w)], idx_v)

        def issue_gather(j):
            b = j % NBUF
            ids = idx_v.at[pl.ds(j * CHUNK, CHUNK)]
            return (
                pltpu.async_copy(cos_hbm.at[ids], cos_v.at[b], gsem[b]),
                pltpu.async_copy(sin_hbm.at[ids], sin_v.at[b], gsem[b]),
            )

        pending_g = [None] * NBUF
        pending_w = [None] * NBUF
        for j in range(min(NBUF - 1, chunks_per_w)):
            pending_g[j % NBUF] = issue_gather(j)
        for j in range(chunks_per_w):
            b = j % NBUF
            jn = j + NBUF - 1
            if jn < chunks_per_w:
                nb = jn % NBUF
                if pending_w[nb] is not None:
                    for d in pending_w[nb]:
                        d.wait()
                    pending_w[nb] = None
                pending_g[nb] = issue_gather(jn)
            for d in pending_g[b]:
                d.wait()
            pending_g[b] = None
            base = row0 + j * CHUNK
            pending_w[b] = (
                pltpu.async_copy(cos_v.at[b], cos_out.at[pl.ds(base, CHUNK)],
                                 wsem[b]),
                pltpu.async_copy(sin_v.at[b], sin_out.at[pl.ds(base, CHUNK)],
                                 wsem[b]),
            )
        for w in pending_w:
            if w is not None:
                for d in w:
                    d.wait()

    return body


def kernel(position_ids, cos_emb, sin_emb):
    B, S = position_ids.shape
    info = plsc.get_sparse_core_info()
    NC, NS = info.num_cores, info.num_subcores
    idx = position_ids.astype(jnp.int32)
    cos_flat, sin_flat = _rope_gather_fn(B, S, NC, NS)(idx, cos_emb, sin_emb)
    return (cos_flat.reshape(B, S, HEAD_DIM),
            sin_flat.reshape(B, S, HEAD_DIM))
